# Initial kernel scaffold; baseline (speedup 1.0000x reference)
import jax
import jax.numpy as jnp
from jax.experimental import pallas as pl


def kernel(x, edge_index, W, b, Wp, bp):
    raise NotImplementedError("write your pallas kernel here")


# speedup vs baseline: 3.4430x; 3.4430x over previous
import functools

import jax
import jax.numpy as jnp
from jax import lax
from jax.experimental import pallas as pl
from jax.experimental.pallas import tpu as pltpu
from jax.experimental.pallas import tpu_sc as plsc

N = 10000
E = 320000
F = 128

NUM_WORKERS = 32
CHUNK = 128
CHUNKS_PER_WORKER = 79
EDGES_PER_WORKER = CHUNK * CHUNKS_PER_WORKER
E_PAD = EDGES_PER_WORKER * NUM_WORKERS
NPAD = 10240
ROWS_PER_TILE = NPAD // 16
R = 1000
GRID = N // R


def _mesh():
    return plsc.VectorSubcoreMesh(core_axis_name="c", subcore_axis_name="s")


@functools.partial(
    pl.kernel,
    mesh=_mesh(),
    out_type=jax.ShapeDtypeStruct((2 * NPAD,), jnp.float32),
    scratch_types=[
        pltpu.VMEM((CHUNK,), jnp.int32),
        pltpu.VMEM((CHUNK,), jnp.float32),
        pltpu.VMEM((ROWS_PER_TILE,), jnp.float32),
        pltpu.VMEM_SHARED((NPAD,), jnp.float32),
    ],
)
def _deg_kernel(dst_hbm, out_hbm, idx_v, ones_v, stage_v, acc_sh):
    c = lax.axis_index("c")
    s = lax.axis_index("s")
    wid = s * 2 + c

    ones16 = jnp.ones((16,), jnp.float32)
    zeros16 = jnp.zeros((16,), jnp.float32)

    def fill_ones(i, carry):
        ones_v[pl.ds(i * 16, 16)] = ones16
        return carry

    lax.fori_loop(0, CHUNK // 16, fill_ones, 0)

    def fill_zero(i, carry):
        stage_v[pl.ds(i * 16, 16)] = zeros16
        return carry

    lax.fori_loop(0, ROWS_PER_TILE // 16, fill_zero, 0)

    my_rows = pl.multiple_of(s * ROWS_PER_TILE, 8)
    pltpu.sync_copy(stage_v, acc_sh.at[pl.ds(my_rows, ROWS_PER_TILE)])
    plsc.subcore_barrier()

    base = wid * EDGES_PER_WORKER

    def body(j, carry):
        off = pl.multiple_of(base + j * CHUNK, 8)
        pltpu.sync_copy(dst_hbm.at[pl.ds(off, CHUNK)], idx_v)
        pltpu.sync_copy(ones_v, acc_sh.at[idx_v], add=True)
        return carry

    lax.fori_loop(0, CHUNKS_PER_WORKER, body, 0)
    plsc.subcore_barrier()

    pltpu.sync_copy(acc_sh.at[pl.ds(my_rows, ROWS_PER_TILE)], stage_v)
    out_off = pl.multiple_of(c * NPAD + my_rows, 8)
    pltpu.sync_copy(stage_v, out_hbm.at[pl.ds(out_off, ROWS_PER_TILE)])


@functools.partial(
    pl.kernel,
    mesh=_mesh(),
    out_type=jax.ShapeDtypeStruct((2 * NPAD, F), jnp.float32),
    scratch_types=[
        pltpu.VMEM((CHUNK,), jnp.int32),
        pltpu.VMEM((CHUNK,), jnp.int32),
        pltpu.VMEM((CHUNK, F), jnp.float32),
        pltpu.VMEM_SHARED((NPAD, F), jnp.float32),
        pltpu.SemaphoreType.DMA,
    ],
)
def _scatter_kernel(tab_hbm, src_hbm, dst_hbm, out_hbm,
                    src_v, dst_v, rows_v, acc_sh, sem):
    c = lax.axis_index("c")
    s = lax.axis_index("s")
    wid = s * 2 + c

    zeros16 = jnp.zeros((16,), jnp.float32)

    def zrow(i, carry):
        for j in range(F // 16):
            rows_v[i, pl.ds(j * 16, 16)] = zeros16
        return carry

    lax.fori_loop(0, CHUNK, zrow, 0)

    my_rows = s * ROWS_PER_TILE

    def zblk(t, carry):
        r0 = pl.multiple_of(my_rows + t * CHUNK, 8)
        pltpu.sync_copy(rows_v, acc_sh.at[pl.ds(r0, CHUNK)])
        return carry

    lax.fori_loop(0, ROWS_PER_TILE // CHUNK, zblk, 0)
    plsc.subcore_barrier()

    base = wid * EDGES_PER_WORKER

    def body(j, carry):
        off = pl.multiple_of(base + j * CHUNK, 8)
        pltpu.sync_copy(src_hbm.at[pl.ds(off, CHUNK)], src_v)
        pltpu.sync_copy(dst_hbm.at[pl.ds(off, CHUNK)], dst_v)
        pltpu.async_copy(tab_hbm.at[src_v], rows_v, sem).wait()
        pltpu.sync_copy(rows_v, acc_sh.at[dst_v], add=True)
        return carry

    lax.fori_loop(0, CHUNKS_PER_WORKER, body, 0)
    plsc.subcore_barrier()

    def wb(t, carry):
        r0 = pl.multiple_of(my_rows + t * CHUNK, 8)
        pltpu.sync_copy(acc_sh.at[pl.ds(r0, CHUNK)], rows_v)
        o0 = pl.multiple_of(c * NPAD + my_rows + t * CHUNK, 8)
        pltpu.sync_copy(rows_v, out_hbm.at[pl.ds(o0, CHUNK)])
        return carry

    lax.fori_loop(0, ROWS_PER_TILE // CHUNK, wb, 0)


def _prep_body(degp_ref, x_ref, dinv_ref, g_ref):
    deg = degp_ref[0] + degp_ref[1]
    dinv = lax.rsqrt(jnp.maximum(deg, 1.0))
    dinv_ref[...] = dinv
    g_ref[...] = x_ref[...] * dinv


def _prep(degp, x):
    return pl.pallas_call(
        _prep_body,
        grid=(GRID,),
        in_specs=[
            pl.BlockSpec((2, R, 1), lambda i: (0, i, 0)),
            pl.BlockSpec((R, F), lambda i: (i, 0)),
        ],
        out_specs=[
            pl.BlockSpec((R, 1), lambda i: (i, 0)),
            pl.BlockSpec((R, F), lambda i: (i, 0)),
        ],
        out_shape=[
            jax.ShapeDtypeStruct((N, 1), jnp.float32),
            jax.ShapeDtypeStruct((N, F), jnp.float32),
        ],
    )(degp, x)


def _mid_body(p_ref, dinv_ref, x1_ref, y2_ref):
    a1 = p_ref[0] + p_ref[1]
    dinv = dinv_ref[...]
    x1 = -dinv * a1
    x1_ref[...] = x1
    y2_ref[...] = dinv * x1


def _mid(p, dinv):
    return pl.pallas_call(
        _mid_body,
        grid=(GRID,),
        in_specs=[
            pl.BlockSpec((2, R, F), lambda i: (0, i, 0)),
            pl.BlockSpec((R, 1), lambda i: (i, 0)),
        ],
        out_specs=[
            pl.BlockSpec((R, F), lambda i: (i, 0)),
            pl.BlockSpec((R, F), lambda i: (i, 0)),
        ],
        out_shape=[
            jax.ShapeDtypeStruct((N, F), jnp.float32),
            jax.ShapeDtypeStruct((N, F), jnp.float32),
        ],
    )(p, dinv)


def _final_body(x_ref, x1_ref, q_ref, dinv_ref, w_ref, b_ref, wp_ref, bp_ref,
                out_ref):
    a2 = q_ref[0] + q_ref[1]
    x = x_ref[...]
    x2 = -2.0 * dinv_ref[...] * a2 - x
    dot = functools.partial(jnp.dot, preferred_element_type=jnp.float32,
                            precision=lax.Precision.HIGHEST)
    h = (dot(x, w_ref[pl.ds(0, F), :])
         + dot(x1_ref[...], w_ref[pl.ds(F, F), :])
         + dot(x2, w_ref[pl.ds(2 * F, F), :])
         + b_ref[...])
    h = jnp.maximum(h, 0.0)
    out_ref[...] = dot(h, wp_ref[...]) + bp_ref[...]


def _final(x, x1, q, dinv, w, b2, wp, bp2):
    return pl.pallas_call(
        _final_body,
        grid=(GRID,),
        in_specs=[
            pl.BlockSpec((R, F), lambda i: (i, 0)),
            pl.BlockSpec((R, F), lambda i: (i, 0)),
            pl.BlockSpec((2, R, F), lambda i: (0, i, 0)),
            pl.BlockSpec((R, 1), lambda i: (i, 0)),
            pl.BlockSpec((3 * F, F), lambda i: (0, 0)),
            pl.BlockSpec((1, F), lambda i: (0, 0)),
            pl.BlockSpec((F, 1), lambda i: (0, 0)),
            pl.BlockSpec((1, 1), lambda i: (0, 0)),
        ],
        out_specs=pl.BlockSpec((R, 1), lambda i: (i, 0)),
        out_shape=jax.ShapeDtypeStruct((N, 1), jnp.float32),
    )(x, x1, q, dinv, w, b2, wp, bp2)


def kernel(x, edge_index, W, b, Wp, bp):
    src = edge_index[0]
    dst = edge_index[1]
    pad = E_PAD - E
    src_p = jnp.concatenate([src, jnp.zeros((pad,), jnp.int32)])
    dst_p = jnp.concatenate([dst, jnp.full((pad,), N, jnp.int32)])

    degp = _deg_kernel(dst_p).reshape(2, NPAD, 1)
    dinv, g = _prep(degp, x)
    p = _scatter_kernel(g, src_p, dst_p).reshape(2, NPAD, F)
    x1, y2 = _mid(p, dinv)
    q = _scatter_kernel(y2, src_p, dst_p).reshape(2, NPAD, F)
    return _final(x, x1, q, dinv, W, b.reshape(1, F), Wp, bp.reshape(1, 1))


# speedup vs baseline: 3.5220x; 1.0229x over previous
import functools

import jax
import jax.numpy as jnp
from jax import lax
from jax.experimental import pallas as pl
from jax.experimental.pallas import tpu as pltpu
from jax.experimental.pallas import tpu_sc as plsc

N = 10000
E = 320000
F = 128

NUM_WORKERS = 32
CHUNK = 128
CHUNKS_PER_WORKER = 80
EDGES_PER_WORKER = CHUNK * CHUNKS_PER_WORKER
E_PAD = EDGES_PER_WORKER * NUM_WORKERS
NPAD = 10240
ROWS_PER_TILE = NPAD // 16
NACC = 10112
ACC_PER_TILE = NACC // 16
R = 1000
GRID = N // R
NB = 3
NG = 2


def _mesh():
    return plsc.VectorSubcoreMesh(core_axis_name="c", subcore_axis_name="s")


@functools.partial(
    pl.kernel,
    mesh=_mesh(),
    out_type=jax.ShapeDtypeStruct((2 * NPAD,), jnp.float32),
    scratch_types=[
        pltpu.VMEM((CHUNKS_PER_WORKER, CHUNK), jnp.int32),
        pltpu.VMEM((CHUNK,), jnp.float32),
        pltpu.VMEM((ROWS_PER_TILE,), jnp.float32),
        pltpu.VMEM_SHARED((NPAD,), jnp.float32),
    ],
)
def _deg_kernel(dstr_hbm, out_hbm, idx_v, ones_v, stage_v, acc_sh):
    c = lax.axis_index("c")
    s = lax.axis_index("s")
    wid = s * 2 + c

    ones16 = jnp.ones((16,), jnp.float32)
    zeros16 = jnp.zeros((16,), jnp.float32)

    def fill_ones(i, carry):
        ones_v[pl.ds(i * 16, 16)] = ones16
        return carry

    lax.fori_loop(0, CHUNK // 16, fill_ones, 0)

    def fill_zero(i, carry):
        stage_v[pl.ds(i * 16, 16)] = zeros16
        return carry

    lax.fori_loop(0, ROWS_PER_TILE // 16, fill_zero, 0)

    my_rows = pl.multiple_of(s * ROWS_PER_TILE, 8)
    pltpu.sync_copy(stage_v, acc_sh.at[pl.ds(my_rows, ROWS_PER_TILE)])
    plsc.subcore_barrier()

    row0 = wid * CHUNKS_PER_WORKER
    pltpu.sync_copy(dstr_hbm.at[pl.ds(row0, CHUNKS_PER_WORKER)], idx_v)

    def body(j, carry):
        pltpu.sync_copy(ones_v, acc_sh.at[idx_v.at[j]], add=True)
        return carry

    lax.fori_loop(0, CHUNKS_PER_WORKER, body, 0)
    plsc.subcore_barrier()

    pltpu.sync_copy(acc_sh.at[pl.ds(my_rows, ROWS_PER_TILE)], stage_v)
    out_off = pl.multiple_of(c * NPAD + my_rows, 8)
    pltpu.sync_copy(stage_v, out_hbm.at[pl.ds(out_off, ROWS_PER_TILE)])


@functools.partial(
    pl.kernel,
    mesh=_mesh(),
    out_type=jax.ShapeDtypeStruct((2 * NACC, F), jnp.float32),
    scratch_types=[
        pltpu.VMEM((NB, 2, CHUNK), jnp.int32),
        pltpu.VMEM((NB, CHUNK, F), jnp.float32),
        pltpu.VMEM_SHARED((NACC, F), jnp.float32),
    ] + [pltpu.SemaphoreType.DMA] * (2 * NB),
)
def _scatter_kernel(tab_hbm, ec_hbm, out_hbm, idx_v, rows_v, acc_sh, *sems):
    isem = sems[:NB]
    gsem = sems[NB:]
    c = lax.axis_index("c")
    s = lax.axis_index("s")
    wid = s * 2 + c

    zeros16 = jnp.zeros((16,), jnp.float32)

    def zrow(i, carry):
        for j in range(F // 16):
            rows_v[0, i, pl.ds(j * 16, 16)] = zeros16
        return carry

    lax.fori_loop(0, CHUNK, zrow, 0)

    my_rows = s * ACC_PER_TILE

    def zblk(t, carry):
        pltpu.sync_copy(rows_v.at[0], acc_sh.at[pl.ds(my_rows + t * CHUNK,
                                                      CHUNK)])
        return carry

    lax.fori_loop(0, ACC_PER_TILE // CHUNK, zblk, 0)
    pltpu.sync_copy(rows_v.at[0, pl.ds(0, ACC_PER_TILE % CHUNK)],
                    acc_sh.at[pl.ds(my_rows + (ACC_PER_TILE // CHUNK) * CHUNK,
                                    ACC_PER_TILE % CHUNK)])
    plsc.subcore_barrier()

    row0 = wid * CHUNKS_PER_WORKER

    for b in range(NB):
        pltpu.async_copy(ec_hbm.at[row0 + b], idx_v.at[b], isem[b])
    for b in range(NG):
        pltpu.make_async_copy(ec_hbm.at[row0 + b], idx_v.at[b],
                              isem[b]).wait()
        pltpu.async_copy(tab_hbm.at[idx_v.at[b, 0]], rows_v.at[b], gsem[b])

    def group(jj, carry):
        for b in range(NB):
            j = jj * NB + b

            @pl.when(j < CHUNKS_PER_WORKER)
            def _():
                pltpu.make_async_copy(tab_hbm.at[idx_v.at[b, 0]],
                                      rows_v.at[b], gsem[b]).wait()
                pltpu.sync_copy(rows_v.at[b], acc_sh.at[idx_v.at[b, 1]],
                                add=True)

            jn_i = j + NB
            bn_i = b

            @pl.when(jn_i < CHUNKS_PER_WORKER)
            def _():
                pltpu.async_copy(ec_hbm.at[row0 + jn_i], idx_v.at[bn_i],
                                 isem[bn_i])

            jn_g = j + NG
            bn_g = (b + NG) % NB

            @pl.when(jn_g < CHUNKS_PER_WORKER)
            def _():
                pltpu.make_async_copy(ec_hbm.at[row0 + jn_g],
                                      idx_v.at[bn_g], isem[bn_g]).wait()
                pltpu.async_copy(tab_hbm.at[idx_v.at[bn_g, 0]],
                                 rows_v.at[bn_g], gsem[bn_g])
        return carry

    lax.fori_loop(0, (CHUNKS_PER_WORKER + NB - 1) // NB, group, 0)
    plsc.subcore_barrier()

    def wb(t, carry):
        r0 = my_rows + t * CHUNK
        pltpu.sync_copy(acc_sh.at[pl.ds(r0, CHUNK)], rows_v.at[0])
        pltpu.sync_copy(rows_v.at[0], out_hbm.at[pl.ds(c * NACC + r0, CHUNK)])
        return carry

    lax.fori_loop(0, ACC_PER_TILE // CHUNK, wb, 0)
    _tail = ACC_PER_TILE % CHUNK
    _t0 = my_rows + (ACC_PER_TILE // CHUNK) * CHUNK
    pltpu.sync_copy(acc_sh.at[pl.ds(_t0, _tail)],
                    rows_v.at[0, pl.ds(0, _tail)])
    pltpu.sync_copy(rows_v.at[0, pl.ds(0, _tail)],
                    out_hbm.at[pl.ds(c * NACC + _t0, _tail)])


def _prep_body(degp_ref, x_ref, dinv_ref, g_ref):
    deg = degp_ref[0] + degp_ref[1]
    dinv = lax.rsqrt(jnp.maximum(deg, 1.0))
    dinv_ref[...] = dinv
    g_ref[...] = x_ref[...] * dinv


def _prep(degp, x):
    return pl.pallas_call(
        _prep_body,
        grid=(GRID,),
        in_specs=[
            pl.BlockSpec((2, R, 1), lambda i: (0, i, 0)),
            pl.BlockSpec((R, F), lambda i: (i, 0)),
        ],
        out_specs=[
            pl.BlockSpec((R, 1), lambda i: (i, 0)),
            pl.BlockSpec((R, F), lambda i: (i, 0)),
        ],
        out_shape=[
            jax.ShapeDtypeStruct((N, 1), jnp.float32),
            jax.ShapeDtypeStruct((N, F), jnp.float32),
        ],
    )(degp, x)


def _mid_body(p_ref, dinv_ref, x1_ref, y2_ref):
    a1 = p_ref[0] + p_ref[1]
    dinv = dinv_ref[...]
    x1 = -dinv * a1
    x1_ref[...] = x1
    y2_ref[...] = dinv * x1


def _mid(p, dinv):
    return pl.pallas_call(
        _mid_body,
        grid=(GRID,),
        in_specs=[
            pl.BlockSpec((2, R, F), lambda i: (0, i, 0)),
            pl.BlockSpec((R, 1), lambda i: (i, 0)),
        ],
        out_specs=[
            pl.BlockSpec((R, F), lambda i: (i, 0)),
            pl.BlockSpec((R, F), lambda i: (i, 0)),
        ],
        out_shape=[
            jax.ShapeDtypeStruct((N, F), jnp.float32),
            jax.ShapeDtypeStruct((N, F), jnp.float32),
        ],
    )(p, dinv)


def _final_body(x_ref, x1_ref, q_ref, dinv_ref, w_ref, b_ref, wp_ref, bp_ref,
                out_ref):
    a2 = q_ref[0] + q_ref[1]
    x = x_ref[...]
    x2 = -2.0 * dinv_ref[...] * a2 - x
    dot = functools.partial(jnp.dot, preferred_element_type=jnp.float32,
                            precision=lax.Precision.HIGHEST)
    h = (dot(x, w_ref[pl.ds(0, F), :])
         + dot(x1_ref[...], w_ref[pl.ds(F, F), :])
         + dot(x2, w_ref[pl.ds(2 * F, F), :])
         + b_ref[...])
    h = jnp.maximum(h, 0.0)
    out_ref[...] = dot(h, wp_ref[...]) + bp_ref[...]


def _final(x, x1, q, dinv, w, b2, wp, bp2):
    return pl.pallas_call(
        _final_body,
        grid=(GRID,),
        in_specs=[
            pl.BlockSpec((R, F), lambda i: (i, 0)),
            pl.BlockSpec((R, F), lambda i: (i, 0)),
            pl.BlockSpec((2, R, F), lambda i: (0, i, 0)),
            pl.BlockSpec((R, 1), lambda i: (i, 0)),
            pl.BlockSpec((3 * F, F), lambda i: (0, 0)),
            pl.BlockSpec((1, F), lambda i: (0, 0)),
            pl.BlockSpec((F, 1), lambda i: (0, 0)),
            pl.BlockSpec((1, 1), lambda i: (0, 0)),
        ],
        out_specs=pl.BlockSpec((R, 1), lambda i: (i, 0)),
        out_shape=jax.ShapeDtypeStruct((N, 1), jnp.float32),
    )(x, x1, q, dinv, w, b2, wp, bp2)


def kernel(x, edge_index, W, b, Wp, bp):
    src = edge_index[0]
    dst = edge_index[1]
    pad = E_PAD - E
    src_p = jnp.concatenate([src, jnp.zeros((pad,), jnp.int32)])
    dst_p = jnp.concatenate([dst, jnp.full((pad,), N, jnp.int32)])
    src_r = src_p.reshape(NUM_WORKERS * CHUNKS_PER_WORKER, CHUNK)
    dst_r = dst_p.reshape(NUM_WORKERS * CHUNKS_PER_WORKER, CHUNK)
    ec = jnp.stack([src_r, dst_r], axis=1)

    degp = _deg_kernel(dst_r).reshape(2, NPAD, 1)
    dinv, g = _prep(degp, x)
    p = _scatter_kernel(g, ec).reshape(2, NACC, F)
    x1, y2 = _mid(p, dinv)
    q = _scatter_kernel(y2, ec).reshape(2, NACC, F)
    return _final(x, x1, q, dinv, W, b.reshape(1, F), Wp, bp.reshape(1, 1))
